# Initial kernel scaffold; baseline (speedup 1.0000x reference)
import jax
import jax.numpy as jnp
from jax.experimental import pallas as pl


def kernel(x, edge_index, c1_W1, c1_W2, c1_Wp, c1_b, c2_W1, c2_W2, c2_Wp, c2_b, ai_W, ai_a, an_W, an_a, d_W, d_b):
    raise NotImplementedError("write your pallas kernel here")


# speedup vs baseline: 177.2586x; 177.2586x over previous
import functools

import jax
import jax.numpy as jnp
from jax import lax
from jax.experimental import pallas as pl
from jax.experimental.pallas import tpu as pltpu
from jax.experimental.pallas import tpu_sc as plsc

_N = 100000
_E = 1600000
_B = 2
_F = 11
_BF = _B * _F
_D = 32
_ONE_COL = _BF
_NC = 2
_NS = 16
_CHUNK = 512
_SUB = 128


@functools.lru_cache(maxsize=None)
def _make_agg(n, e_pad, chunk, sub):
    hn = n // _NC
    ept = e_pad // _NS
    nch = ept // chunk
    nsub = chunk // sub
    wpt = -(-(hn // _NS) // 8) * 8
    accr = _NS * wpt
    mesh = plsc.VectorSubcoreMesh(
        core_axis_name="c", subcore_axis_name="s",
        num_cores=_NC, num_subcores=_NS)

    @functools.partial(
        pl.kernel, mesh=mesh,
        out_type=(jax.ShapeDtypeStruct((n, _D), jnp.float32),
                  jax.ShapeDtypeStruct((n, _D), jnp.float32)),
        scratch_types=[
            pltpu.VMEM_SHARED((accr, _D), jnp.float32),
            pltpu.VMEM((chunk,), jnp.int32),
            pltpu.VMEM((chunk,), jnp.int32),
            pltpu.VMEM((nsub, sub), jnp.int32),
            pltpu.VMEM((chunk, _D), jnp.float32),
            pltpu.SemaphoreType.DMA,
            pltpu.SemaphoreType.DMA,
        ],
        compiler_params=pltpu.CompilerParams(use_tc_tiling_on_sc=False))
    def agg(x0, x1, src_h, dst_h, out0, out1,
            acc, src_v, dst_v, dstl, rows, sem_g, sem_s):
        c = lax.axis_index("c").astype(jnp.int32)
        s = lax.axis_index("s").astype(jnp.int32)
        base = c * jnp.int32(hn)

        for xt, outh in ((x0, out0), (x1, out1)):
            def zrow(i, carry):
                z = jnp.zeros((16,), jnp.float32)
                for q in range(_D // 16):
                    rows[i, pl.ds(q * 16, 16)] = z
                return carry
            lax.fori_loop(jnp.int32(0), jnp.int32(chunk), zrow, jnp.int32(0))
            off = 0
            while off < wpt:
                sz = min(chunk, wpt - off)
                pltpu.sync_copy(rows.at[pl.ds(0, sz)],
                                acc.at[pl.ds(s * jnp.int32(wpt) + off, sz)])
                off += sz
            plsc.subcore_barrier()

            def chunk_body(k, carry):
                e0 = s * jnp.int32(ept) + k * jnp.int32(chunk)
                cp_s = pltpu.async_copy(src_h.at[pl.ds(e0, chunk)], src_v,
                                        sem_g)
                cp_d = pltpu.async_copy(dst_h.at[pl.ds(e0, chunk)], dst_v,
                                        sem_g)
                cp_s.wait()
                cp_d.wait()
                gds = []
                for j in range(nsub):
                    gds.append(pltpu.async_copy(
                        xt.at[src_v.at[pl.ds(j * sub, sub)]],
                        rows.at[pl.ds(j * sub, sub)], sem_g))
                for j in range(nsub):
                    for q in range(sub // 16):
                        dv = dst_v[pl.ds(j * sub + q * 16, 16)]
                        loc = dv - base
                        ok = (loc >= 0) & (loc < hn)
                        dstl[jnp.int32(j), pl.ds(q * 16, 16)] = jnp.where(
                            ok, loc, jnp.int32(hn))
                for d in gds:
                    d.wait()
                sds = []
                for j in range(nsub):
                    sds.append(pltpu.async_copy(
                        rows.at[pl.ds(j * sub, sub)],
                        acc.at[dstl.at[jnp.int32(j)]], sem_s, add=True))
                for d in sds:
                    d.wait()
                return carry
            lax.fori_loop(jnp.int32(0), jnp.int32(nch), chunk_body, jnp.int32(0))
            plsc.subcore_barrier()
            w0 = pl.multiple_of(
                jnp.minimum(s * jnp.int32(wpt), jnp.int32(hn - wpt)), 8)
            pltpu.sync_copy(acc.at[pl.ds(w0, wpt)],
                            outh.at[pl.ds(c * jnp.int32(hn) + w0, wpt)])
            plsc.subcore_barrier()

    return agg


@functools.lru_cache(maxsize=None)
def _make_dense(n, bn, k2):
    nb = n // bn

    def body(agg_ref, x_ref, w1_ref, w2_ref, wp_ref, b_ref, out_ref):
        a = agg_ref[0]
        xv = x_ref[0]
        deg = jnp.maximum(a[:, _ONE_COL:_ONE_COL + 1], 1.0)
        an = a / deg
        h = jnp.tanh(
            jnp.dot(an, w1_ref[...], preferred_element_type=jnp.float32)
            + jnp.dot(xv, w2_ref[...], preferred_element_type=jnp.float32))
        o = jnp.tanh(
            jnp.dot(h, wp_ref[...], preferred_element_type=jnp.float32)
            + b_ref[...])
        col = lax.broadcasted_iota(jnp.int32, (bn, _D), 1)
        out_ref[0] = jnp.where(col == _ONE_COL, 1.0, o)

    return pl.pallas_call(
        body,
        grid=(2, nb),
        in_specs=[
            pl.BlockSpec((1, bn, _D), lambda g, i: (g, i, 0)),
            pl.BlockSpec((1, bn, _D), lambda g, i: (g, i, 0)),
            pl.BlockSpec((_D, k2), lambda g, i: (0, 0)),
            pl.BlockSpec((_D, k2), lambda g, i: (0, 0)),
            pl.BlockSpec((k2, _D), lambda g, i: (0, 0)),
            pl.BlockSpec((1, _D), lambda g, i: (0, 0)),
        ],
        out_specs=pl.BlockSpec((1, bn, _D), lambda g, i: (g, i, 0)),
        out_shape=jax.ShapeDtypeStruct((2, n, _D), jnp.float32),
    )


@functools.lru_cache(maxsize=None)
def _make_attn(n, bn):
    nb = n // bn
    hf = 3 * _F
    ncols = 2 * hf

    def body(x_ref, wc_ref, ac_ref, dw_ref, db_ref, out_ref, accn, accz):
        i = pl.program_id(0)
        for g in range(2):
            xg = x_ref[g]
            sf = jnp.tanh(jnp.dot(xg, wc_ref[g],
                                  preferred_element_type=jnp.float32))
            sw = sf * ac_ref[pl.ds(g, 1), :]
            cols = [jnp.sum(sw[:, hb * _F:(hb + 1) * _F], axis=1,
                            keepdims=True) for hb in range(6)]
            sv = jnp.concatenate(cols, axis=1)
            p = jnp.exp(sv)
            z = jnp.sum(p, axis=0, keepdims=True)
            num = lax.dot_general(p, xg, (((0,), (0,)), ((), ())),
                                  preferred_element_type=jnp.float32)

            @pl.when(i == 0)
            def _init():
                accn[g, pl.ds(0, 6), pl.ds(0, _D)] = num
                accz[g, pl.ds(0, 1), pl.ds(0, 6)] = z

            @pl.when(i > 0)
            def _add():
                accn[g, pl.ds(0, 6), pl.ds(0, _D)] = (
                    accn[g, pl.ds(0, 6), pl.ds(0, _D)] + num)
                accz[g, pl.ds(0, 1), pl.ds(0, 6)] = (
                    accz[g, pl.ds(0, 1), pl.ds(0, 6)] + z)

        @pl.when(i == nb - 1)
        def _fin():
            res = []
            for b in range(2):
                val = db_ref[...]
                for g in range(2):
                    for h in range(3):
                        hb = h * 2 + b
                        v = accn[g, pl.ds(hb, 1), pl.ds(b * _F, _F)]
                        zz = accz[g, pl.ds(0, 1), pl.ds(hb, 1)]
                        w = dw_ref[pl.ds(0, 1),
                                   pl.ds(g * hf + h * _F, _F)]
                        val = val + jnp.sum(v * w, axis=1,
                                            keepdims=True) / zz
                res.append(val)
            out_ref[...] = jnp.concatenate(res, axis=1)

    return pl.pallas_call(
        body,
        grid=(nb,),
        in_specs=[
            pl.BlockSpec((2, bn, _D), lambda i: (0, i, 0)),
            pl.BlockSpec((2, _D, ncols), lambda i: (0, 0, 0)),
            pl.BlockSpec((2, ncols), lambda i: (0, 0)),
            pl.BlockSpec((1, ncols), lambda i: (0, 0)),
            pl.BlockSpec((1, 1), lambda i: (0, 0)),
        ],
        out_specs=pl.BlockSpec((1, 2), lambda i: (0, 0)),
        out_shape=jax.ShapeDtypeStruct((1, 2), jnp.float32),
        scratch_shapes=[pltpu.VMEM((2, 8, 128), jnp.float32),
                        pltpu.VMEM((2, 8, 128), jnp.float32)],
    )


def _big_weights(W1, W2, Wp, b):
    f32 = jnp.float32
    eye = jnp.eye(2, dtype=f32)
    k = W1.shape[1]
    w1b = jnp.concatenate(
        [jnp.kron(eye, W1.astype(f32)),
         jnp.zeros((_D - _BF, 2 * k), f32)], axis=0)
    w2b = jnp.concatenate(
        [jnp.kron(eye, W2.astype(f32)),
         jnp.zeros((_D - _BF, 2 * k), f32)], axis=0)
    wpb = jnp.concatenate(
        [jnp.kron(eye, Wp.astype(f32)),
         jnp.zeros((2 * k, _D - _BF), f32)], axis=1)
    bp = jnp.concatenate(
        [b.astype(f32), b.astype(f32),
         jnp.zeros((_D - _BF,), f32)]).reshape(1, _D)
    return w1b, w2b, wpb, bp


def _att_weights(ai_W, ai_a, an_W, an_a):
    f32 = jnp.float32
    wc = jnp.zeros((2, _D, 66), f32)
    acs = []
    for g, (W, a) in enumerate(((ai_W, ai_a), (an_W, an_a))):
        for h in range(3):
            for b in range(2):
                hb = h * 2 + b
                wc = wc.at[g, b * _F:(b + 1) * _F,
                           hb * _F:(hb + 1) * _F].set(W[h].astype(f32))
        acs.append(jnp.concatenate(
            [a[h].astype(f32) for h in range(3) for _ in range(2)]))
    ac = jnp.stack(acs)
    return wc, ac


def kernel(x, edge_index, c1_W1, c1_W2, c1_Wp, c1_b,
           c2_W1, c2_W2, c2_Wp, c2_b, ai_W, ai_a, an_W, an_a, d_W, d_b):
    with jax.enable_x64(False):
        out = _run(x, edge_index, c1_W1, c1_W2, c1_Wp, c1_b,
                   c2_W1, c2_W2, c2_Wp, c2_b,
                   ai_W, ai_a, an_W, an_a, d_W, d_b)
    return out.astype(jnp.result_type(jnp.float32, jnp.float64))


def _run(x, edge_index, c1_W1, c1_W2, c1_Wp, c1_b,
         c2_W1, c2_W2, c2_Wp, c2_b, ai_W, ai_a, an_W, an_a, d_W, d_b):
    f32 = jnp.float32
    ept = -(-_E // (_NS * _CHUNK)) * _CHUNK
    e_pad = ept * _NS
    src = jnp.concatenate([edge_index[0].astype(jnp.int32),
                           jnp.zeros((e_pad - _E,), jnp.int32)])
    dst = jnp.concatenate([edge_index[1].astype(jnp.int32),
                           jnp.full((e_pad - _E,), _N, jnp.int32)])
    xr = x.astype(f32).reshape(2, _N, _BF)
    xp = jnp.concatenate(
        [xr, jnp.ones((2, _N, 1), f32),
         jnp.zeros((2, _N, _D - _BF - 1), f32)], axis=2)

    agg = _make_agg(_N, e_pad, _CHUNK, _SUB)
    dense32 = _make_dense(_N, 2000, 64)
    dense64 = _make_dense(_N, 2000, 128)
    attn = _make_attn(_N, 2000)

    w1b1, w2b1, wpb1, bp1 = _big_weights(c1_W1, c1_W2, c1_Wp, c1_b)
    w1b2, w2b2, wpb2, bp2 = _big_weights(c2_W1, c2_W2, c2_Wp, c2_b)
    wc, ac = _att_weights(ai_W, ai_a, an_W, an_a)
    dwr = d_W.astype(f32).reshape(66, 1).T
    dbr = d_b.astype(f32).reshape(1, 1)

    a0, a1 = agg(xp[0], xp[1], src, dst)
    h1 = dense32(jnp.stack([a0, a1]), xp, w1b1, w2b1, wpb1, bp1)
    a0, a1 = agg(h1[0], h1[1], src, dst)
    h2 = dense64(jnp.stack([a0, a1]), h1, w1b2, w2b2, wpb2, bp2)
    out = attn(h2, wc, ac, dwr, dbr)
    return out[0]


# speedup vs baseline: 182.9334x; 1.0320x over previous
import functools

import jax
import jax.numpy as jnp
from jax import lax
from jax.experimental import pallas as pl
from jax.experimental.pallas import tpu as pltpu
from jax.experimental.pallas import tpu_sc as plsc

_N = 100000
_E = 1600000
_B = 2
_F = 11
_BF = _B * _F
_D = 24
_ONE_COL = _BF
_NC = 2
_NS = 16
_CHUNK = 512
_SUB = 128


_P = 4
_DEP = 3


@functools.lru_cache(maxsize=None)
def _make_agg(n, e_pad, chunk, sub):
    hn = n // _NC
    ept = e_pad // _NS
    nch = ept // chunk
    nsub = chunk // sub
    assert nch % _P == 0 and nch // _P >= 2
    wpt = -(-(hn // _NS) // 8) * 8
    accr = _NS * wpt
    mesh = plsc.VectorSubcoreMesh(
        core_axis_name="c", subcore_axis_name="s",
        num_cores=_NC, num_subcores=_NS)

    @functools.partial(
        pl.kernel, mesh=mesh,
        out_type=(jax.ShapeDtypeStruct((n, _D), jnp.float32),
                  jax.ShapeDtypeStruct((n, _D), jnp.float32)),
        scratch_types=[
            pltpu.VMEM_SHARED((accr, _D), jnp.float32),
            pltpu.VMEM((_P * chunk,), jnp.int32),
            pltpu.VMEM((_P * chunk,), jnp.int32),
            pltpu.VMEM((_P * nsub, sub), jnp.int32),
            pltpu.VMEM((_P * chunk, _D), jnp.float32),
        ] + [pltpu.SemaphoreType.DMA] * (3 * _P),
        compiler_params=pltpu.CompilerParams(use_tc_tiling_on_sc=False))
    def agg(x0, x1, src_h, dst_h, out0, out1,
            acc, src_v, dst_v, dstl, rows, *sems):
        sem_i = sems[0:_P]
        sem_g = sems[_P:2 * _P]
        sem_s = sems[2 * _P:3 * _P]
        c = lax.axis_index("c").astype(jnp.int32)
        s = lax.axis_index("s").astype(jnp.int32)
        base = c * jnp.int32(hn)

        for xt, outh in ((x0, out0), (x1, out1)):
            def idx_issue(b, kq):
                e0 = s * jnp.int32(ept) + kq * jnp.int32(chunk)
                pltpu.async_copy(src_h.at[pl.ds(e0, chunk)],
                                 src_v.at[pl.ds(b * chunk, chunk)], sem_i[b])
                pltpu.async_copy(dst_h.at[pl.ds(e0, chunk)],
                                 dst_v.at[pl.ds(b * chunk, chunk)], sem_i[b])

            def idx_wait(b):
                for ref in (src_v, dst_v):
                    pltpu.make_async_copy(
                        src_h.at[pl.ds(jnp.int32(0), chunk)],
                        ref.at[pl.ds(b * chunk, chunk)], sem_i[b]).wait()

            def gather_issue(b):
                for j in range(nsub):
                    pltpu.async_copy(
                        xt.at[src_v.at[pl.ds(b * chunk + j * sub, sub)]],
                        rows.at[pl.ds(b * chunk + j * sub, sub)], sem_g[b])

            def gather_wait(b):
                for j in range(nsub):
                    pltpu.make_async_copy(
                        xt.at[src_v.at[pl.ds(b * chunk + j * sub, sub)]],
                        rows.at[pl.ds(b * chunk + j * sub, sub)],
                        sem_g[b]).wait()

            def dstl_compute(b):
                for j in range(nsub):
                    for q in range(sub // 16):
                        dv = dst_v[pl.ds(b * chunk + j * sub + q * 16, 16)]
                        loc = dv - base
                        ok = (loc >= 0) & (loc < hn)
                        dstl[jnp.int32(b * nsub + j), pl.ds(q * 16, 16)] = (
                            jnp.where(ok, loc, jnp.int32(hn)))

            def scatter_issue(b):
                for j in range(nsub):
                    pltpu.async_copy(
                        rows.at[pl.ds(b * chunk + j * sub, sub)],
                        acc.at[dstl.at[jnp.int32(b * nsub + j)]],
                        sem_s[b], add=True)

            def scatter_wait(b):
                for j in range(nsub):
                    pltpu.make_async_copy(
                        rows.at[pl.ds(b * chunk + j * sub, sub)],
                        acc.at[dstl.at[jnp.int32(b * nsub + j)]],
                        sem_s[b]).wait()

            def zrow(i, carry):
                z = jnp.zeros((16,), jnp.float32)
                for q0 in list(range(0, _D - 16, 16)) + [_D - 16]:
                    rows[i, pl.ds(q0, 16)] = z
                return carry
            lax.fori_loop(jnp.int32(0), jnp.int32(chunk), zrow, jnp.int32(0))
            off = 0
            while off < wpt:
                sz = min(chunk, wpt - off)
                pltpu.sync_copy(rows.at[pl.ds(0, sz)],
                                acc.at[pl.ds(s * jnp.int32(wpt) + off, sz)])
                off += sz
            plsc.subcore_barrier()

            idx_issue(0, jnp.int32(0))
            for k in range(_P):
                if k >= _DEP:
                    gather_wait((k - _DEP) % _P)
                    scatter_issue((k - _DEP) % _P)
                idx_issue((k + 1) % _P, jnp.int32(k + 1))
                idx_wait(k)
                gather_issue(k)
                dstl_compute(k)

            def round_body(r, carry):
                k0 = r * jnp.int32(_P)
                for ph in range(_P):
                    scatter_wait(ph)
                    gather_wait((ph - _DEP) % _P)
                    scatter_issue((ph - _DEP) % _P)
                    idx_issue((ph + 1) % _P,
                              jnp.minimum(k0 + jnp.int32(ph + 1),
                                          jnp.int32(nch - 1)))
                    idx_wait(ph)
                    gather_issue(ph)
                    dstl_compute(ph)
                return carry
            lax.fori_loop(jnp.int32(1), jnp.int32(nch // _P), round_body,
                          jnp.int32(0))

            for t in range(_DEP):
                scatter_wait((nch + t) % _P)
                gather_wait((nch + t - _DEP) % _P)
                scatter_issue((nch + t - _DEP) % _P)
            for t in range(_P - _DEP):
                scatter_wait((nch + _DEP + t) % _P)
            idx_wait(nch % _P)

            plsc.subcore_barrier()
            w0 = pl.multiple_of(
                jnp.minimum(s * jnp.int32(wpt), jnp.int32(hn - wpt)), 8)
            pltpu.sync_copy(acc.at[pl.ds(w0, wpt)],
                            outh.at[pl.ds(c * jnp.int32(hn) + w0, wpt)])
            plsc.subcore_barrier()

    return agg


@functools.lru_cache(maxsize=None)
def _make_dense(n, bn, k2):
    nb = n // bn

    def body(agg_ref, x_ref, w1_ref, w2_ref, wp_ref, b_ref, out_ref):
        a = agg_ref[0]
        xv = x_ref[0]
        deg = jnp.maximum(a[:, _ONE_COL:_ONE_COL + 1], 1.0)
        an = a / deg
        h = jnp.tanh(
            jnp.dot(an, w1_ref[...], preferred_element_type=jnp.float32)
            + jnp.dot(xv, w2_ref[...], preferred_element_type=jnp.float32))
        o = jnp.tanh(
            jnp.dot(h, wp_ref[...], preferred_element_type=jnp.float32)
            + b_ref[...])
        col = lax.broadcasted_iota(jnp.int32, (bn, _D), 1)
        out_ref[0] = jnp.where(col == _ONE_COL, 1.0, o)

    return pl.pallas_call(
        body,
        grid=(2, nb),
        in_specs=[
            pl.BlockSpec((1, bn, _D), lambda g, i: (g, i, 0)),
            pl.BlockSpec((1, bn, _D), lambda g, i: (g, i, 0)),
            pl.BlockSpec((_D, k2), lambda g, i: (0, 0)),
            pl.BlockSpec((_D, k2), lambda g, i: (0, 0)),
            pl.BlockSpec((k2, _D), lambda g, i: (0, 0)),
            pl.BlockSpec((1, _D), lambda g, i: (0, 0)),
        ],
        out_specs=pl.BlockSpec((1, bn, _D), lambda g, i: (g, i, 0)),
        out_shape=jax.ShapeDtypeStruct((2, n, _D), jnp.float32),
    )


@functools.lru_cache(maxsize=None)
def _make_attn(n, bn):
    nb = n // bn
    hf = 3 * _F
    ncols = 2 * hf

    def body(x_ref, wc_ref, ac_ref, dw_ref, db_ref, out_ref, accn, accz):
        i = pl.program_id(0)
        for g in range(2):
            xg = x_ref[g]
            sf = jnp.tanh(jnp.dot(xg, wc_ref[g],
                                  preferred_element_type=jnp.float32))
            sw = sf * ac_ref[pl.ds(g, 1), :]
            cols = [jnp.sum(sw[:, hb * _F:(hb + 1) * _F], axis=1,
                            keepdims=True) for hb in range(6)]
            sv = jnp.concatenate(cols, axis=1)
            p = jnp.exp(sv)
            z = jnp.sum(p, axis=0, keepdims=True)
            num = lax.dot_general(p, xg, (((0,), (0,)), ((), ())),
                                  preferred_element_type=jnp.float32)

            @pl.when(i == 0)
            def _init():
                accn[g, pl.ds(0, 6), pl.ds(0, _D)] = num
                accz[g, pl.ds(0, 1), pl.ds(0, 6)] = z

            @pl.when(i > 0)
            def _add():
                accn[g, pl.ds(0, 6), pl.ds(0, _D)] = (
                    accn[g, pl.ds(0, 6), pl.ds(0, _D)] + num)
                accz[g, pl.ds(0, 1), pl.ds(0, 6)] = (
                    accz[g, pl.ds(0, 1), pl.ds(0, 6)] + z)

        @pl.when(i == nb - 1)
        def _fin():
            res = []
            for b in range(2):
                val = db_ref[...]
                for g in range(2):
                    for h in range(3):
                        hb = h * 2 + b
                        v = accn[g, pl.ds(hb, 1), pl.ds(b * _F, _F)]
                        zz = accz[g, pl.ds(0, 1), pl.ds(hb, 1)]
                        w = dw_ref[pl.ds(0, 1),
                                   pl.ds(g * hf + h * _F, _F)]
                        val = val + jnp.sum(v * w, axis=1,
                                            keepdims=True) / zz
                res.append(val)
            out_ref[...] = jnp.concatenate(res, axis=1)

    return pl.pallas_call(
        body,
        grid=(nb,),
        in_specs=[
            pl.BlockSpec((2, bn, _D), lambda i: (0, i, 0)),
            pl.BlockSpec((2, _D, ncols), lambda i: (0, 0, 0)),
            pl.BlockSpec((2, ncols), lambda i: (0, 0)),
            pl.BlockSpec((1, ncols), lambda i: (0, 0)),
            pl.BlockSpec((1, 1), lambda i: (0, 0)),
        ],
        out_specs=pl.BlockSpec((1, 2), lambda i: (0, 0)),
        out_shape=jax.ShapeDtypeStruct((1, 2), jnp.float32),
        scratch_shapes=[pltpu.VMEM((2, 8, 128), jnp.float32),
                        pltpu.VMEM((2, 8, 128), jnp.float32)],
    )


def _big_weights(W1, W2, Wp, b):
    f32 = jnp.float32
    eye = jnp.eye(2, dtype=f32)
    k = W1.shape[1]
    w1b = jnp.concatenate(
        [jnp.kron(eye, W1.astype(f32)),
         jnp.zeros((_D - _BF, 2 * k), f32)], axis=0)
    w2b = jnp.concatenate(
        [jnp.kron(eye, W2.astype(f32)),
         jnp.zeros((_D - _BF, 2 * k), f32)], axis=0)
    wpb = jnp.concatenate(
        [jnp.kron(eye, Wp.astype(f32)),
         jnp.zeros((2 * k, _D - _BF), f32)], axis=1)
    bp = jnp.concatenate(
        [b.astype(f32), b.astype(f32),
         jnp.zeros((_D - _BF,), f32)]).reshape(1, _D)
    return w1b, w2b, wpb, bp


def _att_weights(ai_W, ai_a, an_W, an_a):
    f32 = jnp.float32
    wc = jnp.zeros((2, _D, 66), f32)
    acs = []
    for g, (W, a) in enumerate(((ai_W, ai_a), (an_W, an_a))):
        for h in range(3):
            for b in range(2):
                hb = h * 2 + b
                wc = wc.at[g, b * _F:(b + 1) * _F,
                           hb * _F:(hb + 1) * _F].set(W[h].astype(f32))
        acs.append(jnp.concatenate(
            [a[h].astype(f32) for h in range(3) for _ in range(2)]))
    ac = jnp.stack(acs)
    return wc, ac


def kernel(x, edge_index, c1_W1, c1_W2, c1_Wp, c1_b,
           c2_W1, c2_W2, c2_Wp, c2_b, ai_W, ai_a, an_W, an_a, d_W, d_b):
    with jax.enable_x64(False):
        out = _run(x, edge_index, c1_W1, c1_W2, c1_Wp, c1_b,
                   c2_W1, c2_W2, c2_Wp, c2_b,
                   ai_W, ai_a, an_W, an_a, d_W, d_b)
    return out.astype(jnp.result_type(jnp.float32, jnp.float64))


def _run(x, edge_index, c1_W1, c1_W2, c1_Wp, c1_b,
         c2_W1, c2_W2, c2_Wp, c2_b, ai_W, ai_a, an_W, an_a, d_W, d_b):
    f32 = jnp.float32
    ept = -(-_E // (_NS * _CHUNK * _P)) * _CHUNK * _P
    e_pad = ept * _NS
    src = jnp.concatenate([edge_index[0].astype(jnp.int32),
                           jnp.zeros((e_pad - _E,), jnp.int32)])
    dst = jnp.concatenate([edge_index[1].astype(jnp.int32),
                           jnp.full((e_pad - _E,), _N, jnp.int32)])
    xr = x.astype(f32).reshape(2, _N, _BF)
    xp = jnp.concatenate(
        [xr, jnp.ones((2, _N, 1), f32),
         jnp.zeros((2, _N, _D - _BF - 1), f32)], axis=2)

    agg = _make_agg(_N, e_pad, _CHUNK, _SUB)
    dense32 = _make_dense(_N, 2000, 64)
    dense64 = _make_dense(_N, 2000, 128)
    attn = _make_attn(_N, 2000)

    w1b1, w2b1, wpb1, bp1 = _big_weights(c1_W1, c1_W2, c1_Wp, c1_b)
    w1b2, w2b2, wpb2, bp2 = _big_weights(c2_W1, c2_W2, c2_Wp, c2_b)
    wc, ac = _att_weights(ai_W, ai_a, an_W, an_a)
    dwr = d_W.astype(f32).reshape(66, 1).T
    dbr = d_b.astype(f32).reshape(1, 1)

    a0, a1 = agg(xp[0], xp[1], src, dst)
    h1 = dense32(jnp.stack([a0, a1]), xp, w1b1, w2b1, wpb1, bp1)
    a0, a1 = agg(h1[0], h1[1], src, dst)
    h2 = dense64(jnp.stack([a0, a1]), h1, w1b2, w2b2, wpb2, bp2)
    out = attn(h2, wc, ac, dwr, dbr)
    return out[0]
